# baseline (speedup 1.0000x reference)
import functools

import jax
import jax.numpy as jnp
from jax import lax
from jax.experimental import pallas as pl
from jax.experimental.pallas import tpu as pltpu
from jax.experimental.pallas import tpu_sc as plsc

B, SEQ, D = 4, 4096, 160
MARK_AX = 0
OP_ADD = 1
OP_SUB = 2
ALU_LO = 16
ALU_HI = 32
AX_CARRY_LO = 48
AX_CARRY_HI = 64
OUTPUT_LO = 112
OUTPUT_HI = 128

NC, NS, L = 2, 16, 16
NW = NC * NS
TOKENS = B * SEQ
TPT = TOKENS // NW


def _sc_body(x_hbm, out_hbm, buf):
    wid = lax.axis_index("s") * NC + lax.axis_index("c")
    base = wid * TPT * D
    pltpu.sync_copy(x_hbm.at[pl.ds(base, TPT * D)], buf)

    two = jnp.full((L,), 2.0, jnp.float32)

    def group(g, carry):
        rowoff = (g * L + lax.iota(jnp.int32, L)) * D

        def col(c):
            return plsc.load_gather(buf, [rowoff + c])

        def decode(b0):
            acc = jnp.full((L,), 16, jnp.int32)
            for k in range(15, -1, -1):
                acc = jnp.where(col(b0 + k) > 0.5, k, acc)
            return jnp.where(acc == 16, 0, acc)

        a_lo = decode(ALU_LO)
        a_hi = decode(ALU_HI)
        b_lo = decode(AX_CARRY_LO)
        b_hi = decode(AX_CARRY_HI)

        mark = col(MARK_AX) > 0.5
        is_add = col(OP_ADD) > 0.5
        is_sub = jnp.logical_and(jnp.logical_not(is_add), col(OP_SUB) > 0.5)
        active = jnp.logical_and(mark, jnp.logical_or(is_add, is_sub))

        sum_lo = a_lo + b_lo
        add_r_lo = jnp.bitwise_and(sum_lo, 15)
        carry_v = lax.shift_right_arithmetic(sum_lo, 4)
        add_r_hi = jnp.bitwise_and(a_hi + b_hi + carry_v, 15)

        diff_lo = a_lo - b_lo
        sub_r_lo = jnp.bitwise_and(diff_lo, 15)
        borrow = jnp.where(diff_lo < 0, 1, 0)
        sub_r_hi = jnp.bitwise_and(a_hi - b_hi - borrow, 15)

        r_lo = jnp.where(is_add, add_r_lo, sub_r_lo)
        r_hi = jnp.where(is_add, add_r_hi, sub_r_hi)

        plsc.addupdate_scatter(buf, [rowoff + (r_lo + OUTPUT_LO)], two, mask=active)
        plsc.addupdate_scatter(buf, [rowoff + (r_hi + OUTPUT_HI)], two, mask=active)
        return carry

    lax.fori_loop(0, TPT // L, group, 0)
    pltpu.sync_copy(buf, out_hbm.at[pl.ds(base, TPT * D)])


@jax.jit
def kernel(x_bd):
    x = x_bd.reshape(TOKENS * D)
    mesh = plsc.VectorSubcoreMesh(core_axis_name="c", subcore_axis_name="s")
    out = pl.kernel(
        _sc_body,
        out_type=jax.ShapeDtypeStruct((TOKENS * D,), jnp.float32),
        mesh=mesh,
        scratch_types=[pltpu.VMEM((TPT * D,), jnp.float32)],
        compiler_params=pltpu.CompilerParams(needs_layout_passes=False),
    )(x)
    return out.reshape(B, SEQ, D)
